# baseline (speedup 1.0000x reference)
import jax
import jax.numpy as jnp
from jax.experimental import pallas as pl
from jax.experimental.pallas import tpu as pltpu

_N_GROUP = 8
_TOPK_GROUP = 4
_TOP_K = 8
_SCALE = 2.5
_NEG = -1e30


def _routing(sfc, scores_t):
    e, bt = sfc.shape
    spg = e // _N_GROUP

    g3 = sfc.reshape(_N_GROUP, spg, bt)
    m1 = jnp.max(g3, axis=1, keepdims=True)
    m1b = jnp.broadcast_to(m1, g3.shape)
    eq = g3 == m1b
    cnt = jnp.sum(eq.astype(jnp.float32), axis=1, keepdims=True)
    strict = jnp.max(jnp.where(eq, _NEG, g3), axis=1, keepdims=True)
    m2 = jnp.where(cnt >= 2.0, m1, strict)
    gs = m1 + m2

    growf = jax.lax.broadcasted_iota(
        jnp.int32, (_N_GROUP, 1, bt), 0).astype(jnp.float32)
    gidf = (jax.lax.broadcasted_iota(
        jnp.int32, (e, bt), 0) // spg).astype(jnp.float32)
    t8 = gs
    gmask = jnp.zeros((e, bt), dtype=jnp.bool_)
    for _ in range(_TOPK_GROUP):
        m = jnp.max(t8, axis=0, keepdims=True)
        fi = jnp.min(jnp.where(t8 == m, growf, float(_N_GROUP)),
                     axis=0, keepdims=True)
        fi2 = fi.reshape(1, bt)
        gmask = gmask | (gidf == fi2)
        t8 = jnp.where(growf == fi, _NEG, t8)

    tmp = jnp.where(gmask, sfc, 0.0)

    frow = jax.lax.broadcasted_iota(
        jnp.int32, (e, bt), 0).astype(jnp.float32)
    row8 = jax.lax.broadcasted_iota(
        jnp.int32, (_TOP_K, bt), 0).astype(jnp.float32)
    acc_i = jnp.zeros((_TOP_K, bt), dtype=jnp.float32)
    acc_w = jnp.zeros((_TOP_K, bt), dtype=jnp.float32)
    t = tmp
    for k in range(_TOP_K):
        m = jnp.max(t, axis=0, keepdims=True)
        fi = jnp.min(jnp.where(t == m, frow, float(e)),
                     axis=0, keepdims=True)
        acc_i = jnp.where(row8 == float(k), fi, acc_i)
        acc_w = jnp.where(row8 == float(k), m, acc_w)
        t = jnp.where(frow == fi, _NEG, t)

    denom = jnp.sum(acc_w, axis=0, keepdims=True) + 1e-20
    w_out = acc_w * (_SCALE / denom)
    return acc_i.astype(jnp.int32).T, w_out.T


def _gate_kernel(x_ref, w_ref, b_ref, idx_ref, w_out_ref, acc_ref):
    j = pl.program_id(1)
    nj = pl.num_programs(1)
    partial = jax.lax.dot_general(
        w_ref[...], x_ref[...], (((1,), (1,)), ((), ())),
        preferred_element_type=jnp.float32)

    @pl.when(j == 0)
    def _init():
        acc_ref[...] = partial

    @pl.when(j > 0)
    def _accum():
        acc_ref[...] = acc_ref[...] + partial

    @pl.when(j == nj - 1)
    def _finish():
        logits_t = acc_ref[...]
        scores_t = jax.nn.sigmoid(logits_t)
        sfc = scores_t + b_ref[...]
        idx, wts = _routing(sfc, scores_t)
        idx_ref[...] = idx
        w_out_ref[...] = wts


def kernel(hidden_states, weight, e_score_correction_bias):
    bsz, seq, h = hidden_states.shape
    n_experts = weight.shape[0]
    t = bsz * seq
    bt = 4096
    hb = 1024

    x2 = hidden_states.reshape(t, h)
    w = weight.astype(jnp.float32)
    b2 = e_score_correction_bias.reshape(n_experts, 1).astype(jnp.float32)

    idx, wts = pl.pallas_call(
        _gate_kernel,
        grid=(t // bt, h // hb),
        in_specs=[
            pl.BlockSpec((bt, hb), lambda i, j: (i, j)),
            pl.BlockSpec((n_experts, hb), lambda i, j: (0, j)),
            pl.BlockSpec((n_experts, 1), lambda i, j: (0, 0)),
        ],
        out_specs=[
            pl.BlockSpec((bt, _TOP_K), lambda i, j: (i, 0)),
            pl.BlockSpec((bt, _TOP_K), lambda i, j: (i, 0)),
        ],
        out_shape=[
            jax.ShapeDtypeStruct((t, _TOP_K), jnp.int32),
            jax.ShapeDtypeStruct((t, _TOP_K), jnp.float32),
        ],
        scratch_shapes=[pltpu.VMEM((n_experts, bt), jnp.float32)],
        compiler_params=pltpu.CompilerParams(
            dimension_semantics=("arbitrary", "arbitrary"),
        ),
    )(x2, w, b2)
    return idx, wts


# speedup vs baseline: 1.1725x; 1.1725x over previous
import jax
import jax.numpy as jnp
from jax.experimental import pallas as pl
from jax.experimental.pallas import tpu as pltpu

_N_GROUP = 8
_TOPK_GROUP = 4
_TOP_K = 8
_SCALE = 2.5
_NEG = -1e30


def _routing(sfc, scores_t):
    e, bt = sfc.shape
    spg = e // _N_GROUP

    g3 = sfc.reshape(_N_GROUP, spg, bt)
    m1 = jnp.max(g3, axis=1, keepdims=True)
    m1b = jnp.broadcast_to(m1, g3.shape)
    eq = g3 == m1b
    cnt = jnp.sum(eq.astype(jnp.float32), axis=1, keepdims=True)
    strict = jnp.max(jnp.where(eq, _NEG, g3), axis=1, keepdims=True)
    m2 = jnp.where(cnt >= 2.0, m1, strict)
    gs = m1 + m2

    growf = jax.lax.broadcasted_iota(
        jnp.int32, (_N_GROUP, 1, bt), 0).astype(jnp.float32)
    gidf = (jax.lax.broadcasted_iota(
        jnp.int32, (e, bt), 0) // spg).astype(jnp.float32)
    t8 = gs
    gmask = jnp.zeros((e, bt), dtype=jnp.bool_)
    for _ in range(_TOPK_GROUP):
        m = jnp.max(t8, axis=0, keepdims=True)
        fi = jnp.min(jnp.where(t8 == m, growf, float(_N_GROUP)),
                     axis=0, keepdims=True)
        fi2 = fi.reshape(1, bt)
        gmask = gmask | (gidf == fi2)
        t8 = jnp.where(growf == fi, _NEG, t8)

    tmp = jnp.where(gmask, sfc, 0.0)

    frow = jax.lax.broadcasted_iota(
        jnp.int32, (e, bt), 0).astype(jnp.float32)
    row8 = jax.lax.broadcasted_iota(
        jnp.int32, (_TOP_K, bt), 0).astype(jnp.float32)
    acc_i = jnp.zeros((_TOP_K, bt), dtype=jnp.float32)
    acc_w = jnp.zeros((_TOP_K, bt), dtype=jnp.float32)
    t = tmp
    for k in range(_TOP_K):
        m = jnp.max(t, axis=0, keepdims=True)
        fi = jnp.min(jnp.where(t == m, frow, float(e)),
                     axis=0, keepdims=True)
        acc_i = jnp.where(row8 == float(k), fi, acc_i)
        acc_w = jnp.where(row8 == float(k), m, acc_w)
        t = jnp.where(frow == fi, _NEG, t)

    denom = jnp.sum(acc_w, axis=0, keepdims=True) + 1e-20
    w_out = acc_w * (_SCALE / denom)
    return acc_i.astype(jnp.int32).T, w_out.T


def _gate_kernel(xa_ref, xb_ref, w_ref, b_ref, idx_ref, w_out_ref):
    hh = xa_ref.shape[1]
    logits_t = jax.lax.dot_general(
        w_ref[:, :hh], xa_ref[...], (((1,), (1,)), ((), ())),
        preferred_element_type=jnp.float32)
    logits_t = logits_t + jax.lax.dot_general(
        w_ref[:, hh:], xb_ref[...], (((1,), (1,)), ((), ())),
        preferred_element_type=jnp.float32)
    scores_t = jax.nn.sigmoid(logits_t)
    sfc = scores_t + b_ref[...]
    idx, wts = _routing(sfc, scores_t)
    idx_ref[...] = idx
    w_out_ref[...] = wts


def kernel(hidden_states, weight, e_score_correction_bias):
    bsz, seq, h = hidden_states.shape
    n_experts = weight.shape[0]
    t = bsz * seq
    bt = 1024
    hh = h // 2

    x2 = hidden_states.reshape(t, h)
    w = weight.astype(jnp.float32)
    b2 = e_score_correction_bias.reshape(n_experts, 1).astype(jnp.float32)

    idx, wts = pl.pallas_call(
        _gate_kernel,
        grid=(t // bt,),
        in_specs=[
            pl.BlockSpec((bt, hh), lambda i: (i, 0)),
            pl.BlockSpec((bt, hh), lambda i: (i, 1)),
            pl.BlockSpec((n_experts, h), lambda i: (0, 0)),
            pl.BlockSpec((n_experts, 1), lambda i: (0, 0)),
        ],
        out_specs=[
            pl.BlockSpec((bt, _TOP_K), lambda i: (i, 0)),
            pl.BlockSpec((bt, _TOP_K), lambda i: (i, 0)),
        ],
        out_shape=[
            jax.ShapeDtypeStruct((t, _TOP_K), jnp.int32),
            jax.ShapeDtypeStruct((t, _TOP_K), jnp.float32),
        ],
        compiler_params=pltpu.CompilerParams(
            dimension_semantics=("arbitrary",),
        ),
    )(x2, x2, w, b2)
    return idx, wts
